# baseline (speedup 1.0000x reference)
import functools

import jax
import jax.numpy as jnp
from jax import lax
from jax.experimental import pallas as pl
from jax.experimental.pallas import tpu as pltpu
from jax.experimental.pallas import tpu_sc as plsc

B = 4096
L = 20
D = 64
NTOK = B * L
NC = 2
NS = 16
NW = NC * NS
TOK_PER_W = NTOK // NW
IDXROW = 128
NROWS = TOK_PER_W // IDXROW
CHUNK_ROWS = 4
CHUNK_TOK = CHUNK_ROWS * IDXROW
NCHUNK = NROWS // CHUNK_ROWS
NSLOT = 2


def _make_gather():
    mesh = plsc.VectorSubcoreMesh(core_axis_name="c", subcore_axis_name="s")

    @functools.partial(
        pl.kernel,
        mesh=mesh,
        out_type=jax.ShapeDtypeStruct((NW, NROWS, IDXROW, D), jnp.float32),
        compiler_params=pltpu.CompilerParams(use_tc_tiling_on_sc=False),
        scratch_types=[
            pltpu.VMEM((NROWS, IDXROW), jnp.int32),
            pltpu.VMEM((NSLOT, CHUNK_ROWS, IDXROW, D), jnp.float32),
            pltpu.SemaphoreType.DMA,
            pltpu.SemaphoreType.DMA,
            pltpu.SemaphoreType.DMA,
            pltpu.SemaphoreType.DMA,
        ],
    )
    def gather_kernel(idx_hbm, table_hbm, out_hbm, idx_v, rows_v, g0, g1, o0, o1):
        wid = lax.axis_index("s") * NC + lax.axis_index("c")
        pltpu.sync_copy(idx_hbm.at[wid], idx_v)
        gsems = [g0, g1]
        osems = [o0, o1]
        gathers = [None] * NCHUNK
        writes = [None] * NCHUNK

        def fire_gathers(c):
            s = c % NSLOT
            cps = []
            for k in range(CHUNK_ROWS):
                j = c * CHUNK_ROWS + k
                cps.append(
                    pltpu.async_copy(
                        table_hbm.at[idx_v.at[j]],
                        rows_v.at[s, k],
                        gsems[s],
                    )
                )
            gathers[c] = cps

        fire_gathers(0)
        if NCHUNK > 1:
            fire_gathers(1)
        for c in range(NCHUNK):
            s = c % NSLOT
            for cp in gathers[c]:
                cp.wait()
            writes[c] = pltpu.async_copy(
                rows_v.at[s],
                out_hbm.at[wid, pl.ds(c * CHUNK_ROWS, CHUNK_ROWS)],
                osems[s],
            )
            n = c + NSLOT
            if n < NCHUNK:
                writes[c].wait()
                fire_gathers(n)
        for c in range(max(0, NCHUNK - NSLOT), NCHUNK):
            writes[c].wait()

    return gather_kernel


_GATHER = _make_gather()


def kernel(input_ids, table):
    idx = input_ids.reshape(NW, NROWS, IDXROW)
    out = _GATHER(idx, table)
    return out.reshape(B, L, D)


# speedup vs baseline: 1.0027x; 1.0027x over previous
import functools

import jax
import jax.numpy as jnp
from jax import lax
from jax.experimental import pallas as pl
from jax.experimental.pallas import tpu as pltpu
from jax.experimental.pallas import tpu_sc as plsc

B = 4096
L = 20
D = 64
NTOK = B * L
NC = 2
NS = 16
NW = NC * NS
TOK_PER_W = NTOK // NW
IDXROW = 128
NROWS = TOK_PER_W // IDXROW
CHUNK_ROWS = 4
CHUNK_TOK = CHUNK_ROWS * IDXROW
NCHUNK = NROWS // CHUNK_ROWS
NSLOT = 2


def _make_gather():
    mesh = plsc.VectorSubcoreMesh(core_axis_name="c", subcore_axis_name="s")

    @functools.partial(
        pl.kernel,
        mesh=mesh,
        out_type=jax.ShapeDtypeStruct((NW, TOK_PER_W, D), jnp.float32),
        compiler_params=pltpu.CompilerParams(use_tc_tiling_on_sc=False),
        scratch_types=[
            pltpu.VMEM((TOK_PER_W,), jnp.int32),
            pltpu.VMEM((NSLOT, CHUNK_TOK, D), jnp.float32),
            pltpu.SemaphoreType.DMA,
            pltpu.SemaphoreType.DMA,
            pltpu.SemaphoreType.DMA,
            pltpu.SemaphoreType.DMA,
        ],
    )
    def gather_kernel(idx_hbm, table_hbm, out_hbm, idx_v, rows_v, g0, g1, o0, o1):
        wid = lax.axis_index("s") * NC + lax.axis_index("c")
        pltpu.sync_copy(idx_hbm.at[wid], idx_v)
        gsems = [g0, g1]
        osems = [o0, o1]
        gathers = [None] * NCHUNK
        writes = [None] * NCHUNK

        def fire_gathers(c):
            s = c % NSLOT
            gathers[c] = [
                pltpu.async_copy(
                    table_hbm.at[idx_v.at[pl.ds(c * CHUNK_TOK, CHUNK_TOK)]],
                    rows_v.at[s],
                    gsems[s],
                )
            ]

        fire_gathers(0)
        if NCHUNK > 1:
            fire_gathers(1)
        for c in range(NCHUNK):
            s = c % NSLOT
            for cp in gathers[c]:
                cp.wait()
            writes[c] = pltpu.async_copy(
                rows_v.at[s],
                out_hbm.at[wid, pl.ds(c * CHUNK_TOK, CHUNK_TOK)],
                osems[s],
            )
            n = c + NSLOT
            if n < NCHUNK:
                writes[c].wait()
                fire_gathers(n)
        for c in range(max(0, NCHUNK - NSLOT), NCHUNK):
            writes[c].wait()

    return gather_kernel


_GATHER = _make_gather()


def kernel(input_ids, table):
    idx = input_ids.reshape(NW, TOK_PER_W)
    out = _GATHER(idx, table)
    return out.reshape(B, L, D)


# speedup vs baseline: 1.0908x; 1.0878x over previous
import functools

import jax
import jax.numpy as jnp
from jax import lax
from jax.experimental import pallas as pl
from jax.experimental.pallas import tpu as pltpu
from jax.experimental.pallas import tpu_sc as plsc

B = 4096
L = 20
D = 64
DP = 128
NTOK = B * L
NC = 2
NS = 16
NW = NC * NS
TOK_PER_W = NTOK // NW
IDXROW = 128
NROWS = TOK_PER_W // IDXROW
CHUNK_ROWS = 2
CHUNK_TOK = CHUNK_ROWS * IDXROW
NCHUNK = NROWS // CHUNK_ROWS
NSLOT = 2


def _make_gather():
    mesh = plsc.VectorSubcoreMesh(core_axis_name="c", subcore_axis_name="s")

    @functools.partial(
        pl.kernel,
        mesh=mesh,
        out_type=jax.ShapeDtypeStruct((NW, TOK_PER_W, D), jnp.float32),
        compiler_params=pltpu.CompilerParams(use_tc_tiling_on_sc=False),
        scratch_types=[
            pltpu.VMEM((TOK_PER_W,), jnp.int32),
            pltpu.VMEM((NSLOT, CHUNK_TOK, DP), jnp.float32),
            pltpu.SemaphoreType.DMA,
            pltpu.SemaphoreType.DMA,
            pltpu.SemaphoreType.DMA,
            pltpu.SemaphoreType.DMA,
        ],
    )
    def gather_kernel(idx_hbm, table_hbm, out_hbm, idx_v, rows_v, g0, g1, o0, o1):
        wid = lax.axis_index("s") * NC + lax.axis_index("c")
        pltpu.sync_copy(idx_hbm.at[wid], idx_v)
        gsems = [g0, g1]
        osems = [o0, o1]
        gathers = [None] * NCHUNK
        writes = [None] * NCHUNK

        def fire_gathers(c):
            s = c % NSLOT
            gathers[c] = [
                pltpu.async_copy(
                    table_hbm.at[idx_v.at[pl.ds(c * CHUNK_TOK, CHUNK_TOK)]],
                    rows_v.at[s],
                    gsems[s],
                )
            ]

        fire_gathers(0)
        if NCHUNK > 1:
            fire_gathers(1)
        for c in range(NCHUNK):
            s = c % NSLOT
            for cp in gathers[c]:
                cp.wait()
            writes[c] = pltpu.async_copy(
                rows_v.at[s, :, pl.ds(0, D)],
                out_hbm.at[wid, pl.ds(c * CHUNK_TOK, CHUNK_TOK)],
                osems[s],
            )
            n = c + NSLOT
            if n < NCHUNK:
                writes[c].wait()
                fire_gathers(n)
        for c in range(max(0, NCHUNK - NSLOT), NCHUNK):
            writes[c].wait()

    return gather_kernel


_GATHER = _make_gather()


def kernel(input_ids, table):
    idx = input_ids.reshape(NW, TOK_PER_W)
    table_p = jnp.pad(table, ((0, 0), (0, DP - D)))
    out = _GATHER(idx, table_p)
    return out.reshape(B, L, D)
